# baseline (speedup 1.0000x reference)
import functools

import jax
import jax.numpy as jnp
from jax import lax
from jax.experimental import pallas as pl
from jax.experimental.pallas import tpu as pltpu
from jax.experimental.pallas import tpu_sc as plsc

NC = 2
NS = 16
NW = NC * NS

CHUNK = 1024


def _make_wtrans(N: int, D: int):
  n_full = N // CHUNK
  tail = N - n_full * CHUNK
  assert tail % 16 == 0
  lo = n_full // NW
  extra = n_full - lo * NW
  mesh = plsc.VectorSubcoreMesh(core_axis_name="c", subcore_axis_name="s")

  @functools.partial(
      pl.kernel,
      out_type=jax.ShapeDtypeStruct((N, D), jnp.float32),
      mesh=mesh,
      scratch_types=[
          pltpu.VMEM((D, CHUNK), jnp.float32),
          pltpu.VMEM((CHUNK, D), jnp.float32),
      ],
      compiler_params=pltpu.CompilerParams(
          use_tc_tiling_on_sc=False, needs_layout_passes=False),
  )
  def wtrans(wt_hbm, wrow_hbm, slab, tblk):
    wid = lax.axis_index("s") * NC + lax.axis_index("c")
    base = wid * lo + jnp.minimum(wid, extra)
    count = jnp.where(wid < extra, lo + 1, lo)
    lane = lax.iota(jnp.int32, D)
    rots = [(lane + s) % 16 for s in range(16)]

    def transpose_cols(n_cols):
      @pl.loop(0, n_cols, step=16)
      def _(i0):
        for s in range(16):
          ri = rots[s] + i0
          v = plsc.load_gather(slab, [lane, ri])
          plsc.store_scatter(tblk, [ri, lane], v)

    def body(c, carry):
      c0 = (base + c) * CHUNK
      pltpu.sync_copy(wt_hbm.at[:, pl.ds(c0, CHUNK)], slab)
      transpose_cols(CHUNK)
      pltpu.sync_copy(tblk, wrow_hbm.at[pl.ds(c0, CHUNK), :])
      return carry

    lax.fori_loop(0, count, body, 0)

    if tail:
      @pl.when(wid == NW - 1)
      def _():
        c0 = n_full * CHUNK
        pltpu.sync_copy(
            wt_hbm.at[:, pl.ds(c0, tail)], slab.at[:, pl.ds(0, tail)])
        transpose_cols(tail)
        pltpu.sync_copy(
            tblk.at[pl.ds(0, tail), :], wrow_hbm.at[pl.ds(c0, tail), :])

  return wtrans


def _make_lookup(L: int, B: int, N: int, D: int):
  spans = B // CHUNK
  n_items = L * spans
  per_tile = n_items // NW
  assert per_tile % 2 == 0 and per_tile >= 4
  n_pairs = per_tile // 2
  mesh = plsc.VectorSubcoreMesh(core_axis_name="c", subcore_axis_name="s")

  @functools.partial(
      pl.kernel,
      out_type=jax.ShapeDtypeStruct((L * D, B), jnp.float32),
      mesh=mesh,
      scratch_types=[
          pltpu.VMEM((CHUNK,), jnp.int32),
          pltpu.VMEM((CHUNK,), jnp.int32),
          pltpu.VMEM((CHUNK, D), jnp.float32),
          pltpu.VMEM((CHUNK, D), jnp.float32),
          pltpu.VMEM((D, CHUNK), jnp.float32),
          pltpu.VMEM((D, CHUNK), jnp.float32),
          pltpu.SemaphoreType.DMA,
          pltpu.SemaphoreType.DMA,
          pltpu.SemaphoreType.DMA,
          pltpu.SemaphoreType.DMA,
          pltpu.SemaphoreType.DMA,
          pltpu.SemaphoreType.DMA,
      ],
      compiler_params=pltpu.CompilerParams(
          use_tc_tiling_on_sc=False, needs_layout_passes=False),
  )
  def lookup(xt_hbm, w_hbm, out_hbm, idx0, idx1, rows0, rows1, t0, t1,
             si0, si1, sg0, sg1, so0, so1):
    wid = lax.axis_index("s") * NC + lax.axis_index("c")
    item0 = wid * per_tile
    idx_b = (idx0, idx1)
    rows_b = (rows0, rows1)
    t_b = (t0, t1)
    si = (si0, si1)
    sg = (sg0, sg1)
    so = (so0, so1)
    lane = lax.iota(jnp.int32, D)
    rots = [(lane + s) % 16 for s in range(16)]

    def idx_copy(g, slot):
      item = item0 + g
      l = item // spans
      b0 = (item % spans) * CHUNK
      pltpu.async_copy(xt_hbm.at[l, pl.ds(b0, CHUNK)], idx_b[slot], si[slot])

    def fire_gather(slot):
      return pltpu.async_copy(w_hbm.at[idx_b[slot]], rows_b[slot], sg[slot])

    idx_copy(0, 0)
    idx_copy(1, 1)
    pltpu.make_async_copy(
        xt_hbm.at[0, pl.ds(0, CHUNK)], idx_b[0], si[0]).wait()
    fire_gather(0)

    def pair(p, carry):
      for slot in (0, 1):
        g = 2 * p + slot
        other = 1 - slot
        item = item0 + g
        l = item // spans
        b0 = (item % spans) * CHUNK

        pltpu.make_async_copy(
            w_hbm.at[idx_b[slot]], rows_b[slot], sg[slot]).wait()

        @pl.when(g + 2 < per_tile)
        def _():
          idx_copy(g + 2, slot)

        @pl.when(g + 1 < per_tile)
        def _():
          pltpu.make_async_copy(
              xt_hbm.at[0, pl.ds(0, CHUNK)], idx_b[other], si[other]).wait()
          fire_gather(other)

        @pl.when(p > 0)
        def _():
          pltpu.make_async_copy(
              t_b[slot], out_hbm.at[pl.ds(0, D), pl.ds(0, CHUNK)],
              so[slot]).wait()

        rows = rows_b[slot]
        t = t_b[slot]

        @pl.loop(0, CHUNK, step=16)
        def _(i0):
          for s in range(16):
            ri = rots[s] + i0
            v = plsc.load_gather(rows, [ri, lane])
            plsc.store_scatter(t, [lane, ri], v)

        pltpu.async_copy(
            t, out_hbm.at[pl.ds(l * D, D), pl.ds(b0, CHUNK)], so[slot])
      return carry

    lax.fori_loop(0, n_pairs, pair, 0)

    for slot in (0, 1):
      pltpu.make_async_copy(
          t_b[slot], out_hbm.at[pl.ds(0, D), pl.ds(0, CHUNK)],
          so[slot]).wait()

  return lookup


def kernel(x, W):
  B, L = x.shape
  N, D = W.shape
  xt = x.T.astype(jnp.int32)
  w_row = _make_wtrans(N, D)(W.T)
  out2d = _make_lookup(L, B, N, D)(xt, w_row)
  return jnp.transpose(out2d.reshape(L, D, B), (2, 0, 1))


# speedup vs baseline: 2.4094x; 2.4094x over previous
import functools

import jax
import jax.numpy as jnp
from jax import lax
from jax.experimental import pallas as pl
from jax.experimental.pallas import tpu as pltpu
from jax.experimental.pallas import tpu_sc as plsc

NC = 2
NS = 16
NW = NC * NS

CHUNK = 1024


def _make_lookup(L: int, B: int, N: int, D: int):
  spans = B // CHUNK
  n_items = L * spans
  per_tile = n_items // NW
  assert per_tile % 2 == 0 and per_tile >= 4
  n_pairs = per_tile // 2
  blk = CHUNK // 128
  row_words = (B // 128) * 8 * 128
  mesh = plsc.VectorSubcoreMesh(core_axis_name="c", subcore_axis_name="s")

  @functools.partial(
      pl.kernel,
      out_type=jax.ShapeDtypeStruct((L * 2, row_words), jnp.float32),
      mesh=mesh,
      scratch_types=[
          pltpu.VMEM((CHUNK,), jnp.int32),
          pltpu.VMEM((CHUNK,), jnp.int32),
          pltpu.VMEM((CHUNK, D), jnp.float32),
          pltpu.VMEM((CHUNK, D), jnp.float32),
          pltpu.VMEM((D * CHUNK,), jnp.float32),
          pltpu.VMEM((D * CHUNK,), jnp.float32),
          pltpu.SemaphoreType.DMA,
          pltpu.SemaphoreType.DMA,
          pltpu.SemaphoreType.DMA,
          pltpu.SemaphoreType.DMA,
          pltpu.SemaphoreType.DMA,
          pltpu.SemaphoreType.DMA,
      ],
      compiler_params=pltpu.CompilerParams(
          use_tc_tiling_on_sc=False, needs_layout_passes=False),
  )
  def lookup(xt_hbm, w_hbm, out_hbm, idx0, idx1, rows0, rows1, t0, t1,
             si0, si1, sg0, sg1, so0, so1):
    wid = lax.axis_index("s") * NC + lax.axis_index("c")
    item0 = wid * per_tile
    idx_b = (idx0, idx1)
    rows_b = (rows0, rows1)
    t_b = (t0, t1)
    si = (si0, si1)
    sg = (sg0, sg1)
    so = (so0, so1)
    lane = lax.iota(jnp.int32, D)
    rots = [(lane + s) % 16 for s in range(16)]
    drots = [(lane // 8) * (8 * CHUNK) + (lane % 8) * 128 + rots[s]
             for s in range(16)]

    def idx_copy(g, slot):
      item = item0 + g
      l = item // spans
      b0 = (item % spans) * CHUNK
      pltpu.async_copy(xt_hbm.at[l, pl.ds(b0, CHUNK)], idx_b[slot], si[slot])

    def fire_gather(slot):
      return pltpu.async_copy(w_hbm.at[idx_b[slot]], rows_b[slot], sg[slot])

    idx_copy(0, 0)
    idx_copy(1, 1)
    pltpu.make_async_copy(
        xt_hbm.at[0, pl.ds(0, CHUNK)], idx_b[0], si[0]).wait()
    fire_gather(0)

    def drain_out(slot):
      for h in range(2):
        pltpu.make_async_copy(
            t_b[slot].at[pl.ds(h * (8 * CHUNK), 8 * CHUNK)],
            out_hbm.at[0, pl.ds(0, 8 * CHUNK)], so[slot]).wait()

    def pair(p, carry):
      for slot in (0, 1):
        g = 2 * p + slot
        other = 1 - slot
        item = item0 + g
        l = item // spans
        s_i = item % spans

        pltpu.make_async_copy(
            w_hbm.at[idx_b[slot]], rows_b[slot], sg[slot]).wait()

        @pl.when(g + 2 < per_tile)
        def _():
          idx_copy(g + 2, slot)

        @pl.when(g + 1 < per_tile)
        def _():
          pltpu.make_async_copy(
              xt_hbm.at[0, pl.ds(0, CHUNK)], idx_b[other], si[other]).wait()
          fire_gather(other)

        @pl.when(p > 0)
        def _():
          drain_out(slot)

        rows = rows_b[slot]
        t = t_b[slot]

        @pl.loop(0, CHUNK, step=16)
        def _(i0):
          sb = (i0 // 128) * 1024 + i0 % 128
          for s in range(16):
            ri = rots[s] + i0
            v = plsc.load_gather(rows, [ri, lane])
            plsc.store_scatter(t, [drots[s] + sb], v)

        for h in range(2):
          pltpu.async_copy(
              t.at[pl.ds(h * (8 * CHUNK), 8 * CHUNK)],
              out_hbm.at[l * 2 + h, pl.ds(s_i * (8 * CHUNK), 8 * CHUNK)],
              so[slot])
      return carry

    lax.fori_loop(0, n_pairs, pair, 0)

    for slot in (0, 1):
      drain_out(slot)

  return lookup


def kernel(x, W):
  B, L = x.shape
  N, D = W.shape
  xt = x.T.astype(jnp.int32)
  out2d = _make_lookup(L, B, N, D)(xt, W)
  out5 = out2d.reshape(L, 2, B // 128, 8, 128)
  out3 = jnp.transpose(out5, (0, 1, 3, 2, 4)).reshape(L, D, B)
  return jnp.transpose(out3, (2, 0, 1))
